# Initial kernel scaffold; baseline (speedup 1.0000x reference)
import jax
import jax.numpy as jnp
from jax.experimental import pallas as pl


def kernel(pts, params):
    raise NotImplementedError("write your pallas kernel here")


# speedup vs baseline: 8.8200x; 8.8200x over previous
import functools

import jax
import jax.numpy as jnp
from jax import lax
from jax.experimental import pallas as pl
from jax.experimental.pallas import tpu as pltpu

_K = 16


def _silu(x):
    return x * (1.0 / (1.0 + jnp.exp(-x)))


def _knn_prep(xk, xr, R, N):
    sq = jnp.sum(xk * xk, axis=1, keepdims=True)
    sqr = jnp.sum(xr * xr, axis=1, keepdims=True)
    inner = lax.dot_general(xr, xk, (((1,), (1,)), ((), ())),
                            preferred_element_type=jnp.float32)
    d2 = sqr + jnp.transpose(sq) - 2.0 * inner
    return jnp.sqrt(jnp.maximum(d2, 0.0))


def _pick_and_mask(d_ref, iota, N):
    dc = d_ref[...]
    m = jnp.max(dc, axis=1, keepdims=True)
    pick = jnp.min(jnp.where(dc == m, iota, N), axis=1, keepdims=True)
    sel = iota == pick
    d_ref[...] = jnp.where(sel, -jnp.inf, dc)
    return m, sel


def _layer1_body(xk_ref, xwT_ref, fwT_ref, misc_ref,
                 xs_out_ref, f_out_ref, d_ref, *, R, N):
    rb = pl.program_id(1)
    xk = xk_ref[0]
    r0 = pl.multiple_of(rb * R, R)
    xr = xk_ref[0, pl.ds(r0, R), :]
    d_ref[...] = _knn_prep(xk, xr, R, N)
    iota = lax.broadcasted_iota(jnp.int32, (R, N), 1)

    base = misc_ref[0:1, :]
    wl = misc_ref[1:2, :]
    xb = misc_ref[2:3, 0:8]
    fb = misc_ref[3:4, :]

    _pick_and_mask(d_ref, iota, N)

    def body(j, carry):
        s, xacc = carry
        m, sel = _pick_and_mask(d_ref, iota, N)
        oh = sel.astype(jnp.float32)
        xg = jnp.dot(oh, xk, preferred_element_type=jnp.float32)
        xdsq = m * m
        mj = _silu(base + wl * xdsq)
        s = s + mj
        phi = _silu(jnp.dot(mj, xwT_ref[...],
                            preferred_element_type=jnp.float32) + xb)
        xd = xg - xr
        xd3 = xd[:, 0:3]
        zr2 = jnp.zeros((R, 2), jnp.float32)
        xdc = jnp.concatenate([xd3, xd3, zr2], axis=1)
        p0 = phi[:, 0:1]
        p1 = phi[:, 1:2]
        phic = jnp.concatenate([p0, p0, p0, p1, p1, p1, p0, p0], axis=1)
        xacc = xacc + xdc * phic
        return s, xacc

    s0 = jnp.zeros((R, 64), jnp.float32)
    a0 = jnp.zeros((R, 8), jnp.float32)
    s, xacc = lax.fori_loop(0, _K, body, (s0, a0))

    xr3 = xr[:, 0:3]
    zr2 = jnp.zeros((R, 2), jnp.float32)
    xs_base = jnp.concatenate([xr3, xr3, zr2], axis=1)
    xs_out_ref[0] = xs_base + xacc * (1.0 / _K)
    f_out_ref[0] = _silu(jnp.dot(s, fwT_ref[...],
                                 preferred_element_type=jnp.float32) + fb)


def _layer_body(xk_ref, f_ref, ew1T_ref, dwT_ref, xwT_ref, fwT_ref, misc_ref,
                xs_out_ref, f_out_ref, d_ref, *, R, N):
    rb = pl.program_id(1)
    xk = xk_ref[0]
    f = f_ref[0]
    r0 = pl.multiple_of(rb * R, R)
    xr = xk_ref[0, pl.ds(r0, R), :]
    fr = f_ref[0, pl.ds(r0, R), :]
    d_ref[...] = _knn_prep(xk, xr, R, N)
    iota = lax.broadcasted_iota(jnp.int32, (R, N), 1)

    g = jnp.dot(f, ew1T_ref[...], preferred_element_type=jnp.float32)
    hr = jnp.dot(fr, dwT_ref[...], preferred_element_type=jnp.float32)
    table = jnp.concatenate([g, xk], axis=1)

    eb = misc_ref[0:1, :]
    wl = misc_ref[1:2, :]
    xb = misc_ref[2:3, 0:8]
    fb = misc_ref[3:4, :]

    _pick_and_mask(d_ref, iota, N)

    def body(j, carry):
        s, xacc = carry
        m, sel = _pick_and_mask(d_ref, iota, N)
        oh = sel.astype(jnp.float32)
        gath = jnp.dot(oh, table, preferred_element_type=jnp.float32)
        gg = gath[:, 0:64]
        xg = gath[:, 64:72]
        xdsq = m * m
        mj = _silu(gg + hr + wl * xdsq + eb)
        s = s + mj
        phi = _silu(jnp.dot(mj, xwT_ref[...],
                            preferred_element_type=jnp.float32) + xb)
        xd = xg - xr
        p0 = phi[:, 0:1]
        p1 = phi[:, 1:2]
        phic = jnp.concatenate([p0, p0, p0, p1, p1, p1, p0, p0], axis=1)
        xacc = xacc + xd * phic
        return s, xacc

    s0 = jnp.zeros((R, 64), jnp.float32)
    a0 = jnp.zeros((R, 8), jnp.float32)
    s, xacc = lax.fori_loop(0, _K, body, (s0, a0))

    xs_out_ref[0] = xr + xacc * (1.0 / _K)
    f_in = jnp.concatenate([fr, s], axis=1)
    f_out_ref[0] = _silu(jnp.dot(f_in, fwT_ref[...],
                                 preferred_element_type=jnp.float32) + fb)


def _call_layer1(xk, xwT, fwT, misc, R):
    B, N, _ = xk.shape
    body = functools.partial(_layer1_body, R=R, N=N)
    return pl.pallas_call(
        body,
        grid=(B, N // R),
        in_specs=[
            pl.BlockSpec((1, N, 8), lambda b, rb: (b, 0, 0)),
            pl.BlockSpec((64, 8), lambda b, rb: (0, 0)),
            pl.BlockSpec((64, 64), lambda b, rb: (0, 0)),
            pl.BlockSpec((8, 64), lambda b, rb: (0, 0)),
        ],
        out_specs=[
            pl.BlockSpec((1, R, 8), lambda b, rb: (b, rb, 0)),
            pl.BlockSpec((1, R, 64), lambda b, rb: (b, rb, 0)),
        ],
        out_shape=[
            jax.ShapeDtypeStruct((B, N, 8), jnp.float32),
            jax.ShapeDtypeStruct((B, N, 64), jnp.float32),
        ],
        scratch_shapes=[pltpu.VMEM((R, N), jnp.float32)],
        compiler_params=pltpu.CompilerParams(
            dimension_semantics=("parallel", "arbitrary")),
    )(xk, xwT, fwT, misc)


def _call_layer(xk, f, ew1T, dwT, xwT, fwT, misc, R):
    B, N, _ = xk.shape
    body = functools.partial(_layer_body, R=R, N=N)
    return pl.pallas_call(
        body,
        grid=(B, N // R),
        in_specs=[
            pl.BlockSpec((1, N, 8), lambda b, rb: (b, 0, 0)),
            pl.BlockSpec((1, N, 64), lambda b, rb: (b, 0, 0)),
            pl.BlockSpec((64, 64), lambda b, rb: (0, 0)),
            pl.BlockSpec((64, 64), lambda b, rb: (0, 0)),
            pl.BlockSpec((64, 8), lambda b, rb: (0, 0)),
            pl.BlockSpec((128, 64), lambda b, rb: (0, 0)),
            pl.BlockSpec((8, 64), lambda b, rb: (0, 0)),
        ],
        out_specs=[
            pl.BlockSpec((1, R, 8), lambda b, rb: (b, rb, 0)),
            pl.BlockSpec((1, R, 64), lambda b, rb: (b, rb, 0)),
        ],
        out_shape=[
            jax.ShapeDtypeStruct((B, N, 8), jnp.float32),
            jax.ShapeDtypeStruct((B, N, 64), jnp.float32),
        ],
        scratch_shapes=[pltpu.VMEM((R, N), jnp.float32)],
        compiler_params=pltpu.CompilerParams(
            dimension_semantics=("parallel", "arbitrary")),
    )(xk, f, ew1T, dwT, xwT, fwT, misc)


def _pad_cols(a, n):
    return jnp.pad(a, ((0, 0), (0, n - a.shape[1])))


def _misc(row0, row1, xb, fb):
    z = jnp.zeros((64,), jnp.float32)
    xbp = jnp.pad(xb, (0, 64 - xb.shape[0]))
    return jnp.stack([row0, row1, xbp, fb, z, z, z, z], axis=0)


def kernel(pts, params):
    B, _, N = pts.shape
    R = 256 if N % 256 == 0 else N // 2
    p = params

    xk = jnp.transpose(pts[:, :3, :], (0, 2, 1))
    xk = jnp.pad(xk, ((0, 0), (0, 0), (0, 5)))

    misc1 = _misc(p['e1_w'][:, 1] + p['e1_b'], p['e1_w'][:, 2],
                  p['x1_b'], p['f1_w'][:, 0] + p['f1_b'])
    xwT1 = _pad_cols(p['x1_w'].T, 8)
    f1wT = p['f1_w'][:, 1:].T
    xs, f = _call_layer1(xk, xwT1, f1wT, misc1, R)

    for l in (2, 3, 4):
        ew = p[f'e{l}_w']
        ew1T = ew[:, 0:64].T
        dwT = (ew[:, 64:128] - ew[:, 0:64]).T
        xwT = _pad_cols(p[f'x{l}_w'].T, 8)
        fwT = p[f'f{l}_w'].T
        misc = _misc(p[f'e{l}_b'], ew[:, 128], p[f'x{l}_b'], p[f'f{l}_b'])
        xs, f = _call_layer(xs, f, ew1T, dwT, xwT, fwT, misc, R)

    x_out = jnp.transpose(xs[:, :, 0:6], (0, 2, 1))
    f_out = jnp.transpose(f, (0, 2, 1))
    return x_out, f_out
